# baseline (speedup 1.0000x reference)
import functools

import jax
import jax.numpy as jnp
import numpy as np
from jax import lax
from jax.experimental import pallas as pl
from jax.experimental.pallas import tpu as pltpu
from jax.experimental.pallas import tpu_sc as plsc

N = 10000
D = 128
H = 256
E = 320000
Q = 4096

NC = 2
NS = 16
KCH = 128
RT_A = 632
RT_LAST = N - (NS - 1) * RT_A

_BN_INV = 1.0 / np.sqrt(1.0 + 1e-5)


def _sc_mesh():
    return plsc.VectorSubcoreMesh(core_axis_name="c", subcore_axis_name="s",
                                  num_cores=NC, num_subcores=NS)


def _make_sc_agg(feature_split):
    ncht = E // KCH if feature_split else E // NC // KCH
    cpt = ncht // NS
    rem = ncht - cpt * NS
    cmax = cpt + (1 if rem else 0)

    @functools.partial(
        pl.kernel,
        out_type=jax.ShapeDtypeStruct((2 * N, D), jnp.float32),
        mesh=_sc_mesh(),
        scratch_types=[
            pltpu.VMEM((2, 1, 2, KCH), jnp.int32),
            pltpu.VMEM((2, KCH, D), jnp.float32),
            pltpu.VMEM_SHARED((N, D), jnp.float32),
            pltpu.SemaphoreType.DMA,
            pltpu.SemaphoreType.DMA,
        ],
    )
    def k(rows_hbm, echunks_hbm, zeros_hbm, out_hbm,
          eidx_v, rows_v, acc, sem0, sem1):
        sems = (sem0, sem1)
        c = lax.axis_index("c")
        s = lax.axis_index("s")
        r0 = s * RT_A

        @pl.when(s < NS - 1)
        def _():
            pltpu.sync_copy(zeros_hbm, acc.at[pl.ds(r0, RT_A)])

        @pl.when(s == NS - 1)
        def _():
            pltpu.sync_copy(zeros_hbm.at[pl.ds(0, RT_LAST)],
                            acc.at[pl.ds((NS - 1) * RT_A, RT_LAST)])

        nmine = cpt + jnp.where(s < rem, 1, 0)
        cbase = s * cpt + jnp.minimum(s, rem)
        if not feature_split:
            cbase = cbase + c * ncht
        plsc.subcore_barrier()

        def fetch(i, b):
            pltpu.sync_copy(echunks_hbm.at[pl.ds(cbase + i, 1)],
                            eidx_v.at[b])
            if feature_split:
                for j in range(KCH // 16):
                    sl = pl.ds(j * 16, 16)
                    eidx_v[b, 0, 0, sl] = eidx_v[b, 0, 0, sl] + c * N
            pltpu.async_copy(rows_hbm.at[eidx_v.at[b, 0, 0]], rows_v.at[b],
                             sems[b])

        def drain_scatter(b):
            pltpu.make_async_copy(rows_hbm.at[pl.ds(0, KCH)],
                                  rows_v.at[b], sems[b]).wait()
            pltpu.sync_copy(rows_v.at[b], acc.at[eidx_v.at[b, 0, 1]],
                            add=True)

        fetch(0, 0)

        def body(k2, carry):
            i0 = 2 * k2
            fetch(i0 + 1, 1)
            drain_scatter(0)

            @pl.when(i0 + 2 < nmine)
            def _():
                fetch(i0 + 2, 0)

            drain_scatter(1)
            return carry

        lax.fori_loop(0, cpt // 2, body, 0)

        @pl.when(nmine > cpt)
        def _():
            drain_scatter(0)

        plsc.subcore_barrier()

        @pl.when(s < NS - 1)
        def _():
            pltpu.sync_copy(acc.at[pl.ds(r0, RT_A)],
                            out_hbm.at[pl.ds(c * N + r0, RT_A)])

        @pl.when(s == NS - 1)
        def _():
            pltpu.sync_copy(acc.at[pl.ds((NS - 1) * RT_A, RT_LAST)],
                            out_hbm.at[pl.ds(c * N + (NS - 1) * RT_A, RT_LAST)])

    return k


_sc_agg_edges = _make_sc_agg(False)
_sc_agg_feat = _make_sc_agg(True)

QPT = Q // (NC * NS)
_CN_GRP = 4
_XCH = 16

W0 = 3328
WT = 128
NMAIN = 3 * W0
NPC = NMAIN + WT


def _make_sc_cn():
    qpt = Q // (NC * NS)
    ngrp = qpt // 4

    @functools.partial(
        pl.kernel,
        out_type=[
            jax.ShapeDtypeStruct((Q * NPC,), jnp.float32),
            jax.ShapeDtypeStruct((Q, H), jnp.float32),
            jax.ShapeDtypeStruct((Q, H), jnp.float32),
        ],
        mesh=_sc_mesh(),
        scratch_types=[
            pltpu.VMEM((_XCH,), jnp.int32),
            pltpu.VMEM((_XCH, H), jnp.float32),
            pltpu.VMEM((2, 8), jnp.int32),
            pltpu.VMEM((8, W0), jnp.float32),
            pltpu.VMEM((8, W0), jnp.float32),
            pltpu.VMEM((8, WT), jnp.float32),
            pltpu.VMEM((2, 4 * W0), jnp.float32),
            pltpu.VMEM((4 * WT,), jnp.float32),
            pltpu.SemaphoreType.DMA,
            pltpu.SemaphoreType.DMA,
            pltpu.SemaphoreType.DMA,
            pltpu.SemaphoreType.DMA,
            pltpu.SemaphoreType.DMA,
            pltpu.SemaphoreType.DMA,
        ],
    )
    def _sc_cn(adj_hbm, tail_hbm, pairs_hbm, posf_hbm, xl_hbm,
               cn_hbm, xi_hbm, xj_hbm,
               idxx_v, xrows_v, pidx_v, bufA, bufB, bufL,
               st0_v, stL_v,
               semA, semB, semL, wsem0a, wsem0b, semx):
        wsem0 = (wsem0a, wsem0b)
        c = lax.axis_index("c")
        s = lax.axis_index("s")
        wid = s * NC + c
        q0 = wid * qpt
        pbase = q0 * 2

        for t in range(qpt // _XCH):
            qt = q0 + t * _XCH
            pltpu.sync_copy(posf_hbm.at[pl.ds(qt, _XCH)], idxx_v)
            pltpu.async_copy(xl_hbm.at[idxx_v], xrows_v, semx).wait()
            pltpu.sync_copy(xrows_v, xi_hbm.at[pl.ds(qt, _XCH)])
            pltpu.sync_copy(posf_hbm.at[pl.ds(Q + qt, _XCH)], idxx_v)
            pltpu.async_copy(xl_hbm.at[idxx_v], xrows_v, semx).wait()
            pltpu.sync_copy(xrows_v, xj_hbm.at[pl.ds(qt, _XCH)])

        def load_pidx(g, iset):
            pltpu.sync_copy(pairs_hbm.at[pl.ds(pbase + g * 8, 8)],
                            pidx_v.at[iset])

        def fetch_w(iset, buf, sem, coff, w):
            pltpu.async_copy(adj_hbm.at[pidx_v.at[iset], pl.ds(coff, w)],
                             buf, sem)

        def fetch_tail(iset):
            pltpu.async_copy(tail_hbm.at[pidx_v.at[iset]], bufL, semL)

        def products(buf, store):
            w = buf.shape[1]

            def cols(cc, c2):
                for u in range(4):
                    for gg in range(4):
                        co = (cc * 4 + gg) * 16
                        sl = pl.ds(co, 16)
                        store(u, co, buf[2 * u, sl] * buf[2 * u + 1, sl])
                return c2

            lax.fori_loop(0, w // 64, cols, 0)

        def compute_w0(buf, set_, qb, coff, skip_wait):
            wait = pltpu.make_async_copy(
                st0_v.at[set_], cn_hbm.at[pl.ds(0, 4 * W0)], wsem0[set_])
            if skip_wait is None:
                wait.wait()
            else:
                @pl.when(skip_wait)
                def _():
                    wait.wait()
            products(buf, lambda u, co, v: st0_v.__setitem__(
                (set_, pl.ds(u * W0 + co, 16)), v))
            for u in range(4):
                pltpu.async_copy(
                    st0_v.at[set_, pl.ds(u * W0, W0)],
                    cn_hbm.at[pl.ds((qb + u) * NPC + coff, W0)],
                    wsem0[set_])

        def compute_sync(buf, st, qb, coff, w):
            products(buf, lambda u, co, v: st.__setitem__(
                (pl.ds(u * w + co, 16),), v))
            for u in range(4):
                pltpu.sync_copy(st.at[pl.ds(u * w, w)],
                                cn_hbm.at[pl.ds((qb + u) * NPC + coff, w)])

        def drain(buf, sem, coff, w):
            pltpu.make_async_copy(adj_hbm.at[pl.ds(0, 8), pl.ds(coff, w)],
                                  buf, sem).wait()

        def drain_tail():
            pltpu.make_async_copy(tail_hbm.at[pl.ds(0, 8)], bufL, semL).wait()

        load_pidx(0, 0)
        fetch_w(0, bufA, semA, 0, W0)

        def section(g2, g, iset, bA, sA, bB, sB, so, first):
            qb = q0 + g * 4
            fetch_w(iset, bB, sB, W0, W0)
            drain(bA, sA, 0, W0)
            compute_w0(bA, so, qb, 0, (g2 > 0) if first else None)
            fetch_w(iset, bA, sA, 2 * W0, W0)
            drain(bB, sB, W0, W0)
            compute_w0(bB, 1 - so, qb, W0, (g2 > 0) if first else None)
            fetch_tail(iset)
            drain(bA, sA, 2 * W0, W0)
            compute_w0(bA, so, qb, 2 * W0, None)
            return qb

        def body(g2, carry):
            g = 2 * g2
            qb = section(g2, g, 0, bufA, semA, bufB, semB, 0, True)
            load_pidx(g + 1, 1)
            fetch_w(1, bufB, semB, 0, W0)
            drain_tail()
            compute_sync(bufL, stL_v, qb, NMAIN, WT)

            qb = section(g2, g + 1, 1, bufB, semB, bufA, semA, 1, False)

            @pl.when(g2 < ngrp // 2 - 1)
            def _():
                load_pidx(g + 2, 0)
                fetch_w(0, bufA, semA, 0, W0)

            drain_tail()
            compute_sync(bufL, stL_v, qb, NMAIN, WT)
            return carry

        lax.fori_loop(0, ngrp // 2, body, 0)
        for set_ in range(2):
            pltpu.make_async_copy(st0_v.at[set_],
                                  cn_hbm.at[pl.ds(0, 4 * W0)],
                                  wsem0[set_]).wait()

    return _sc_cn


_sc_cn = _make_sc_cn()


def _ln(h, g, b):
    m = jnp.mean(h, axis=-1, keepdims=True)
    v = jnp.mean((h - m) ** 2, axis=-1, keepdims=True)
    return (h - m) * jax.lax.rsqrt(v + 1e-5) * g + b


BN_ = 1000


def _tc_mlp1_body(x_ref, aggp_ref, w1a_ref, b1a_ref, w1b_ref, b1b_ref,
                  g_ref, bb_ref, eps_ref, out_ref):
    h = x_ref[...] * (1.0 + eps_ref[0, 0]) + aggp_ref[0] + aggp_ref[1]
    h = jnp.maximum(jnp.dot(h, w1a_ref[...],
                            preferred_element_type=jnp.float32) + b1a_ref[...], 0.0)
    h = jnp.maximum(jnp.dot(h, w1b_ref[...],
                            preferred_element_type=jnp.float32) + b1b_ref[...], 0.0)
    y = h * (_BN_INV * g_ref[...]) + bb_ref[...]
    out_ref[0] = y[:, :D]
    out_ref[1] = y[:, D:]


def _tc_mlp1(x, aggp, w1a, b1a, w1b, b1b, g, b, eps):
    grid = (N // BN_,)
    return pl.pallas_call(
        _tc_mlp1_body,
        grid=grid,
        in_specs=[
            pl.BlockSpec((BN_, D), lambda i: (i, 0)),
            pl.BlockSpec((2, BN_, D), lambda i: (0, i, 0)),
            pl.BlockSpec((D, H), lambda i: (0, 0)),
            pl.BlockSpec((1, H), lambda i: (0, 0)),
            pl.BlockSpec((H, H), lambda i: (0, 0)),
            pl.BlockSpec((1, H), lambda i: (0, 0)),
            pl.BlockSpec((1, H), lambda i: (0, 0)),
            pl.BlockSpec((1, H), lambda i: (0, 0)),
            pl.BlockSpec((1, 1), lambda i: (0, 0)),
        ],
        out_specs=pl.BlockSpec((2, BN_, D), lambda i: (0, i, 0)),
        out_shape=jax.ShapeDtypeStruct((2, N, D), jnp.float32),
        compiler_params=pltpu.CompilerParams(
            dimension_semantics=("parallel",)),
    )(x, aggp, w1a, b1a, w1b, b1b, g, b, eps)


def _tc_mlp2_body(x1h_ref, a2h_ref, w2a_ref, b2a_ref, g2_ref, bb2_ref,
                  wl1_ref, bl1_ref, wx1_ref, bx1_ref, wx2_ref, bx2_ref,
                  lng_ref, lnb_ref, eps_ref, xl_ref, x3_ref):
    e = 1.0 + eps_ref[0, 0]
    ta = x1h_ref[0] * e + a2h_ref[0]
    tb = x1h_ref[1] * e + a2h_ref[1]
    h = (jnp.dot(ta, w2a_ref[:D, :], preferred_element_type=jnp.float32)
         + jnp.dot(tb, w2a_ref[D:, :], preferred_element_type=jnp.float32)
         + b2a_ref[...])
    h = jnp.maximum(h, 0.0)
    x2 = h * (_BN_INV * g2_ref[...]) + bb2_ref[...]
    xl = jnp.dot(x2, wl1_ref[...], preferred_element_type=jnp.float32) + bl1_ref[...]
    hx = jnp.maximum(jnp.dot(xl, wx1_ref[...],
                             preferred_element_type=jnp.float32) + bx1_ref[...], 0.0)
    hx = jnp.dot(hx, wx2_ref[...], preferred_element_type=jnp.float32) + bx2_ref[...]
    hx = jnp.maximum(_ln(hx, lng_ref[...], lnb_ref[...]), 0.0)
    xl_ref[...] = xl
    x3_ref[...] = xl + hx


def _tc_mlp2(x1h, a2h, w2a, b2a, g2, b2, wl1, bl1, wx1, bx1, wx2, bx2,
             lng, lnb, eps):
    grid = (N // BN_,)
    hh = pl.BlockSpec((H, H), lambda i: (0, 0))
    vh = pl.BlockSpec((1, H), lambda i: (0, 0))
    return pl.pallas_call(
        _tc_mlp2_body,
        grid=grid,
        in_specs=[
            pl.BlockSpec((2, BN_, D), lambda i: (0, i, 0)),
            pl.BlockSpec((2, BN_, D), lambda i: (0, i, 0)),
            hh, vh, vh, vh,
            hh, vh, hh, vh, hh, vh,
            vh, vh,
            pl.BlockSpec((1, 1), lambda i: (0, 0)),
        ],
        out_specs=[
            pl.BlockSpec((BN_, H), lambda i: (i, 0)),
            pl.BlockSpec((BN_, H), lambda i: (i, 0)),
        ],
        out_shape=[
            jax.ShapeDtypeStruct((N, H), jnp.float32),
            jax.ShapeDtypeStruct((N, H), jnp.float32),
        ],
        compiler_params=pltpu.CompilerParams(
            dimension_semantics=("parallel",)),
    )(x1h, a2h, w2a, b2a, g2, b2, wl1, bl1, wx1, bx1, wx2, bx2, lng, lnb, eps)


BQ = 512
BQC = 256


def _tc_cnmm_body(cn_ref, x3_ref, o_ref):
    o_ref[...] = jnp.dot(cn_ref[...], x3_ref[...],
                         preferred_element_type=jnp.float32)


def _tc_cnmm(cn, x3):
    grid = (Q // BQC,)
    return pl.pallas_call(
        _tc_cnmm_body,
        grid=grid,
        in_specs=[
            pl.BlockSpec((BQC, NPC), lambda i: (i, 0)),
            pl.BlockSpec((NPC, H), lambda i: (0, 0)),
        ],
        out_specs=pl.BlockSpec((BQC, H), lambda i: (i, 0)),
        out_shape=jax.ShapeDtypeStruct((Q, H), jnp.float32),
        compiler_params=pltpu.CompilerParams(
            dimension_semantics=("parallel",)),
    )(cn, x3)


def _tc_final_body(xcn_ref, xi_ref, xj_ref,
                   wi1_ref, bi1_ref, lnig_ref, lnib_ref, wi2_ref, bi2_ref,
                   wc1_ref, bc1_ref, wc2_ref, bc2_ref, lncg_ref, lncb_ref,
                   wc3_ref, bc3_ref, beta_ref,
                   wl1_ref, bl1_ref, ln1g_ref, ln1b_ref,
                   wl2_ref, bl2_ref, ln2g_ref, ln2b_ref,
                   wl3_ref, bl3_ref, o_ref):
    dot = lambda a, w, b: jnp.dot(a, w[...],
                                  preferred_element_type=jnp.float32) + b[...]
    hij = dot(xi_ref[...] * xj_ref[...], wi1_ref, bi1_ref)
    hij = jnp.maximum(_ln(hij, lnig_ref[...], lnib_ref[...]), 0.0)
    xij = dot(hij, wi2_ref, bi2_ref)
    hc = jnp.maximum(dot(xcn_ref[...], wc1_ref, bc1_ref), 0.0)
    hc = dot(hc, wc2_ref, bc2_ref)
    hc = jnp.maximum(_ln(hc, lncg_ref[...], lncb_ref[...]), 0.0)
    hc = dot(hc, wc3_ref, bc3_ref)
    pre = hc * beta_ref[0, 0] + xij
    o = dot(pre, wl1_ref, bl1_ref)
    o = jnp.maximum(_ln(o, ln1g_ref[...], ln1b_ref[...]), 0.0)
    o = dot(o, wl2_ref, bl2_ref)
    o = jnp.maximum(_ln(o, ln2g_ref[...], ln2b_ref[...]), 0.0)
    o_ref[...] = dot(o, wl3_ref, bl3_ref)


def _tc_final(xcn, xi, xj, args):
    grid = (Q // BQ,)
    hh = pl.BlockSpec((H, H), lambda i: (0, 0))
    vh = pl.BlockSpec((1, H), lambda i: (0, 0))
    qh = pl.BlockSpec((BQ, H), lambda i: (i, 0))
    return pl.pallas_call(
        _tc_final_body,
        grid=grid,
        in_specs=[
            qh, qh, qh,
            hh, vh, vh, vh, hh, vh,
            hh, vh, hh, vh, vh, vh, hh, vh,
            pl.BlockSpec((1, 1), lambda i: (0, 0)),
            hh, vh, vh, vh,
            hh, vh, vh, vh,
            pl.BlockSpec((H, D), lambda i: (0, 0)),
            pl.BlockSpec((1, D), lambda i: (0, 0)),
        ],
        out_specs=pl.BlockSpec((BQ, D), lambda i: (i, 0)),
        out_shape=jax.ShapeDtypeStruct((Q, D), jnp.float32),
        compiler_params=pltpu.CompilerParams(
            dimension_semantics=("parallel",)),
    )(xcn, xi, xj, *args)


def kernel(x, edge_index, adj, pos_edge, params):
    p = params
    zeros = jnp.zeros((RT_A, D), jnp.float32)
    r2 = lambda v: v.reshape(1, -1)

    echunks = jnp.pad(edge_index.reshape(2, E // KCH, KCH).transpose(1, 0, 2),
                      ((0, 8), (0, 0), (0, 0)))
    aggp = _sc_agg_edges(x, echunks, zeros).reshape(2, N, D)
    x1h = _tc_mlp1(x, aggp, p['W1a'], r2(p['b1a']), p['W1b'], r2(p['b1b']),
                   r2(p['bn1_g']), r2(p['bn1_b']),
                   p['eps1'].reshape(1, 1).astype(jnp.float32))
    x1flat = x1h.reshape(2 * N, D)
    a2h = _sc_agg_feat(x1flat, echunks, zeros).reshape(2, N, D)
    xl, x3 = _tc_mlp2(x1h, a2h, p['W2a'], r2(p['b2a']),
                      r2(p['bn2_g']), r2(p['bn2_b']),
                      p['Wl1'], r2(p['bl1']), p['Wx1'], r2(p['bx1']),
                      p['Wx2'], r2(p['bx2']), r2(p['lnx_g']), r2(p['lnx_b']),
                      p['eps2'].reshape(1, 1).astype(jnp.float32))

    tail = jnp.pad(adj[:, NMAIN:], ((0, 0), (0, WT - (N - NMAIN))))
    pairs = jnp.stack([pos_edge[0], pos_edge[1]], axis=1).ravel()
    posf = pos_edge.ravel()
    x3p = jnp.pad(x3, ((0, NPC - N), (0, 0)))
    cn, xi, xj = _sc_cn(adj, tail, pairs, posf, xl)
    xcn = _tc_cnmm(cn.reshape(Q, NPC), x3p)

    wl3 = jnp.pad(p['WL3'], ((0, 0), (0, D - p['WL3'].shape[1])))
    bl3 = jnp.pad(p['bL3'], (0, D - p['bL3'].shape[0])).reshape(1, D)
    args = (p['Wi1'], r2(p['bi1']), r2(p['lni_g']), r2(p['lni_b']),
            p['Wi2'], r2(p['bi2']),
            p['Wc1'], r2(p['bc1']), p['Wc2'], r2(p['bc2']),
            r2(p['lnc_g']), r2(p['lnc_b']), p['Wc3'], r2(p['bc3']),
            p['beta'].reshape(1, 1).astype(jnp.float32),
            p['WL1'], r2(p['bL1']), r2(p['lnL1_g']), r2(p['lnL1_b']),
            p['WL2'], r2(p['bL2']), r2(p['lnL2_g']), r2(p['lnL2_b']),
            wl3, bl3)
    o = _tc_final(xcn, xi, xj, args)
    return o[:, :7]


# speedup vs baseline: 1.0609x; 1.0609x over previous
import functools

import jax
import jax.numpy as jnp
import numpy as np
from jax import lax
from jax.experimental import pallas as pl
from jax.experimental.pallas import tpu as pltpu
from jax.experimental.pallas import tpu_sc as plsc

N = 10000
D = 128
H = 256
E = 320000
Q = 4096

NC = 2
NS = 16
KCH = 128
RT_A = 632
RT_LAST = N - (NS - 1) * RT_A

_BN_INV = 1.0 / np.sqrt(1.0 + 1e-5)


def _sc_mesh():
    return plsc.VectorSubcoreMesh(core_axis_name="c", subcore_axis_name="s",
                                  num_cores=NC, num_subcores=NS)


def _make_sc_agg(feature_split):
    ncht = E // KCH if feature_split else E // NC // KCH
    cpt = ncht // NS
    rem = ncht - cpt * NS
    cmax = cpt + (1 if rem else 0)

    @functools.partial(
        pl.kernel,
        out_type=jax.ShapeDtypeStruct((2 * N, D), jnp.float32),
        mesh=_sc_mesh(),
        scratch_types=[
            pltpu.VMEM((2, 1, 2, KCH), jnp.int32),
            pltpu.VMEM((2, KCH, D), jnp.float32),
            pltpu.VMEM_SHARED((N, D), jnp.float32),
            pltpu.SemaphoreType.DMA,
            pltpu.SemaphoreType.DMA,
        ],
    )
    def k(rows_hbm, echunks_hbm, zeros_hbm, out_hbm,
          eidx_v, rows_v, acc, sem0, sem1):
        sems = (sem0, sem1)
        c = lax.axis_index("c")
        s = lax.axis_index("s")
        r0 = s * RT_A

        @pl.when(s < NS - 1)
        def _():
            pltpu.sync_copy(zeros_hbm, acc.at[pl.ds(r0, RT_A)])

        @pl.when(s == NS - 1)
        def _():
            pltpu.sync_copy(zeros_hbm.at[pl.ds(0, RT_LAST)],
                            acc.at[pl.ds((NS - 1) * RT_A, RT_LAST)])

        nmine = cpt + jnp.where(s < rem, 1, 0)
        cbase = s * cpt + jnp.minimum(s, rem)
        if not feature_split:
            cbase = cbase + c * ncht
        plsc.subcore_barrier()

        def fetch(i, b):
            pltpu.sync_copy(echunks_hbm.at[pl.ds(cbase + i, 1)],
                            eidx_v.at[b])
            if feature_split:
                for j in range(KCH // 16):
                    sl = pl.ds(j * 16, 16)
                    eidx_v[b, 0, 0, sl] = eidx_v[b, 0, 0, sl] + c * N
            pltpu.async_copy(rows_hbm.at[eidx_v.at[b, 0, 0]], rows_v.at[b],
                             sems[b])

        def drain_scatter(b):
            pltpu.make_async_copy(rows_hbm.at[pl.ds(0, KCH)],
                                  rows_v.at[b], sems[b]).wait()
            pltpu.sync_copy(rows_v.at[b], acc.at[eidx_v.at[b, 0, 1]],
                            add=True)

        fetch(0, 0)

        def body(k2, carry):
            i0 = 2 * k2
            fetch(i0 + 1, 1)
            drain_scatter(0)

            @pl.when(i0 + 2 < nmine)
            def _():
                fetch(i0 + 2, 0)

            drain_scatter(1)
            return carry

        lax.fori_loop(0, cpt // 2, body, 0)

        @pl.when(nmine > cpt)
        def _():
            drain_scatter(0)

        plsc.subcore_barrier()

        @pl.when(s < NS - 1)
        def _():
            pltpu.sync_copy(acc.at[pl.ds(r0, RT_A)],
                            out_hbm.at[pl.ds(c * N + r0, RT_A)])

        @pl.when(s == NS - 1)
        def _():
            pltpu.sync_copy(acc.at[pl.ds((NS - 1) * RT_A, RT_LAST)],
                            out_hbm.at[pl.ds(c * N + (NS - 1) * RT_A, RT_LAST)])

    return k


_sc_agg_edges = _make_sc_agg(False)
_sc_agg_feat = _make_sc_agg(True)

QPT = Q // (NC * NS)
_CN_GRP = 4
_XCH = 16

W0 = 3328
WT = 128
NMAIN = 3 * W0
NPC = NMAIN + WT


def _make_sc_cn():
    qpt = Q // (NC * NS)
    ngrp = qpt // 4

    @functools.partial(
        pl.kernel,
        out_type=[
            jax.ShapeDtypeStruct((Q * NPC,), jnp.float32),
            jax.ShapeDtypeStruct((Q, H), jnp.float32),
            jax.ShapeDtypeStruct((Q, H), jnp.float32),
        ],
        mesh=_sc_mesh(),
        scratch_types=[
            pltpu.VMEM((_XCH,), jnp.int32),
            pltpu.VMEM((_XCH, H), jnp.float32),
            pltpu.VMEM((2, 8), jnp.int32),
            pltpu.VMEM((8, W0), jnp.float32),
            pltpu.VMEM((8, W0), jnp.float32),
            pltpu.VMEM((8, W0), jnp.float32),
            pltpu.VMEM((8, WT), jnp.float32),
            pltpu.VMEM((2, 4 * W0), jnp.float32),
            pltpu.VMEM((4 * WT,), jnp.float32),
            pltpu.SemaphoreType.DMA,
            pltpu.SemaphoreType.DMA,
            pltpu.SemaphoreType.DMA,
            pltpu.SemaphoreType.DMA,
            pltpu.SemaphoreType.DMA,
            pltpu.SemaphoreType.DMA,
            pltpu.SemaphoreType.DMA,
        ],
    )
    def _sc_cn(adj_hbm, tail_hbm, pairs_hbm, posf_hbm, xl_hbm,
               cn_hbm, xi_hbm, xj_hbm,
               idxx_v, xrows_v, pidx_v, bufA, bufB, bufC, bufL,
               st0_v, stL_v,
               semA, semB, semC, semL, wsem0a, wsem0b, semx):
        wsem0 = (wsem0a, wsem0b)
        c = lax.axis_index("c")
        s = lax.axis_index("s")
        wid = s * NC + c
        q0 = wid * qpt
        pbase = q0 * 2

        for t in range(qpt // _XCH):
            qt = q0 + t * _XCH
            pltpu.sync_copy(posf_hbm.at[pl.ds(qt, _XCH)], idxx_v)
            pltpu.async_copy(xl_hbm.at[idxx_v], xrows_v, semx).wait()
            pltpu.sync_copy(xrows_v, xi_hbm.at[pl.ds(qt, _XCH)])
            pltpu.sync_copy(posf_hbm.at[pl.ds(Q + qt, _XCH)], idxx_v)
            pltpu.async_copy(xl_hbm.at[idxx_v], xrows_v, semx).wait()
            pltpu.sync_copy(xrows_v, xj_hbm.at[pl.ds(qt, _XCH)])

        def load_pidx(g, iset):
            pltpu.sync_copy(pairs_hbm.at[pl.ds(pbase + g * 8, 8)],
                            pidx_v.at[iset])

        def fetch_w(iset, buf, sem, coff, w):
            pltpu.async_copy(adj_hbm.at[pidx_v.at[iset], pl.ds(coff, w)],
                             buf, sem)

        def fetch_tail(iset):
            pltpu.async_copy(tail_hbm.at[pidx_v.at[iset]], bufL, semL)

        def products(buf, store):
            w = buf.shape[1]

            def cols(cc, c2):
                for u in range(4):
                    for gg in range(4):
                        co = (cc * 4 + gg) * 16
                        sl = pl.ds(co, 16)
                        store(u, co, buf[2 * u, sl] * buf[2 * u + 1, sl])
                return c2

            lax.fori_loop(0, w // 64, cols, 0)

        def compute_w0(buf, set_, qb, coff, skip_wait):
            wait = pltpu.make_async_copy(
                st0_v.at[set_], cn_hbm.at[pl.ds(0, 4 * W0)], wsem0[set_])
            if skip_wait is None:
                wait.wait()
            else:
                @pl.when(skip_wait)
                def _():
                    wait.wait()
            products(buf, lambda u, co, v: st0_v.__setitem__(
                (set_, pl.ds(u * W0 + co, 16)), v))
            for u in range(4):
                pltpu.async_copy(
                    st0_v.at[set_, pl.ds(u * W0, W0)],
                    cn_hbm.at[pl.ds((qb + u) * NPC + coff, W0)],
                    wsem0[set_])

        def compute_sync(buf, st, qb, coff, w):
            products(buf, lambda u, co, v: st.__setitem__(
                (pl.ds(u * w + co, 16),), v))
            for u in range(4):
                pltpu.sync_copy(st.at[pl.ds(u * w, w)],
                                cn_hbm.at[pl.ds((qb + u) * NPC + coff, w)])

        def drain(buf, sem, coff, w):
            pltpu.make_async_copy(adj_hbm.at[pl.ds(0, 8), pl.ds(coff, w)],
                                  buf, sem).wait()

        def drain_tail():
            pltpu.make_async_copy(tail_hbm.at[pl.ds(0, 8)], bufL, semL).wait()

        load_pidx(0, 0)
        fetch_w(0, bufA, semA, 0, W0)
        fetch_w(0, bufB, semB, W0, W0)

        def section(g2, g, iset, so, first, last):
            qb = q0 + g * 4
            fetch_w(iset, bufC, semC, 2 * W0, W0)
            drain(bufA, semA, 0, W0)
            compute_w0(bufA, so, qb, 0, (g2 > 0) if first else None)
            fetch_tail(iset)
            drain(bufB, semB, W0, W0)
            compute_w0(bufB, 1 - so, qb, W0, (g2 > 0) if first else None)

            nxt = 1 - iset
            if last:
                @pl.when(g2 < ngrp // 2 - 1)
                def _():
                    load_pidx(g + 1, nxt)
                    fetch_w(nxt, bufA, semA, 0, W0)
            else:
                load_pidx(g + 1, nxt)
                fetch_w(nxt, bufA, semA, 0, W0)
            drain(bufC, semC, 2 * W0, W0)
            compute_w0(bufC, so, qb, 2 * W0, None)

            if last:
                @pl.when(g2 < ngrp // 2 - 1)
                def _():
                    fetch_w(nxt, bufB, semB, W0, W0)
            else:
                fetch_w(nxt, bufB, semB, W0, W0)
            drain_tail()
            compute_sync(bufL, stL_v, qb, NMAIN, WT)
            return qb

        def body(g2, carry):
            g = 2 * g2
            section(g2, g, 0, 0, True, False)
            section(g2, g + 1, 1, 1, False, True)
            return carry

        lax.fori_loop(0, ngrp // 2, body, 0)
        for set_ in range(2):
            pltpu.make_async_copy(st0_v.at[set_],
                                  cn_hbm.at[pl.ds(0, 4 * W0)],
                                  wsem0[set_]).wait()

    return _sc_cn


_sc_cn = _make_sc_cn()


def _ln(h, g, b):
    m = jnp.mean(h, axis=-1, keepdims=True)
    v = jnp.mean((h - m) ** 2, axis=-1, keepdims=True)
    return (h - m) * jax.lax.rsqrt(v + 1e-5) * g + b


BN_ = 1000


def _tc_mlp1_body(x_ref, aggp_ref, w1a_ref, b1a_ref, w1b_ref, b1b_ref,
                  g_ref, bb_ref, eps_ref, out_ref):
    h = x_ref[...] * (1.0 + eps_ref[0, 0]) + aggp_ref[0] + aggp_ref[1]
    h = jnp.maximum(jnp.dot(h, w1a_ref[...],
                            preferred_element_type=jnp.float32) + b1a_ref[...], 0.0)
    h = jnp.maximum(jnp.dot(h, w1b_ref[...],
                            preferred_element_type=jnp.float32) + b1b_ref[...], 0.0)
    y = h * (_BN_INV * g_ref[...]) + bb_ref[...]
    out_ref[0] = y[:, :D]
    out_ref[1] = y[:, D:]


def _tc_mlp1(x, aggp, w1a, b1a, w1b, b1b, g, b, eps):
    grid = (N // BN_,)
    return pl.pallas_call(
        _tc_mlp1_body,
        grid=grid,
        in_specs=[
            pl.BlockSpec((BN_, D), lambda i: (i, 0)),
            pl.BlockSpec((2, BN_, D), lambda i: (0, i, 0)),
            pl.BlockSpec((D, H), lambda i: (0, 0)),
            pl.BlockSpec((1, H), lambda i: (0, 0)),
            pl.BlockSpec((H, H), lambda i: (0, 0)),
            pl.BlockSpec((1, H), lambda i: (0, 0)),
            pl.BlockSpec((1, H), lambda i: (0, 0)),
            pl.BlockSpec((1, H), lambda i: (0, 0)),
            pl.BlockSpec((1, 1), lambda i: (0, 0)),
        ],
        out_specs=pl.BlockSpec((2, BN_, D), lambda i: (0, i, 0)),
        out_shape=jax.ShapeDtypeStruct((2, N, D), jnp.float32),
        compiler_params=pltpu.CompilerParams(
            dimension_semantics=("parallel",)),
    )(x, aggp, w1a, b1a, w1b, b1b, g, b, eps)


def _tc_mlp2_body(x1h_ref, a2h_ref, w2a_ref, b2a_ref, g2_ref, bb2_ref,
                  wl1_ref, bl1_ref, wx1_ref, bx1_ref, wx2_ref, bx2_ref,
                  lng_ref, lnb_ref, eps_ref, xl_ref, x3_ref):
    e = 1.0 + eps_ref[0, 0]
    ta = x1h_ref[0] * e + a2h_ref[0]
    tb = x1h_ref[1] * e + a2h_ref[1]
    h = (jnp.dot(ta, w2a_ref[:D, :], preferred_element_type=jnp.float32)
         + jnp.dot(tb, w2a_ref[D:, :], preferred_element_type=jnp.float32)
         + b2a_ref[...])
    h = jnp.maximum(h, 0.0)
    x2 = h * (_BN_INV * g2_ref[...]) + bb2_ref[...]
    xl = jnp.dot(x2, wl1_ref[...], preferred_element_type=jnp.float32) + bl1_ref[...]
    hx = jnp.maximum(jnp.dot(xl, wx1_ref[...],
                             preferred_element_type=jnp.float32) + bx1_ref[...], 0.0)
    hx = jnp.dot(hx, wx2_ref[...], preferred_element_type=jnp.float32) + bx2_ref[...]
    hx = jnp.maximum(_ln(hx, lng_ref[...], lnb_ref[...]), 0.0)
    xl_ref[...] = xl
    x3_ref[...] = xl + hx


def _tc_mlp2(x1h, a2h, w2a, b2a, g2, b2, wl1, bl1, wx1, bx1, wx2, bx2,
             lng, lnb, eps):
    grid = (N // BN_,)
    hh = pl.BlockSpec((H, H), lambda i: (0, 0))
    vh = pl.BlockSpec((1, H), lambda i: (0, 0))
    return pl.pallas_call(
        _tc_mlp2_body,
        grid=grid,
        in_specs=[
            pl.BlockSpec((2, BN_, D), lambda i: (0, i, 0)),
            pl.BlockSpec((2, BN_, D), lambda i: (0, i, 0)),
            hh, vh, vh, vh,
            hh, vh, hh, vh, hh, vh,
            vh, vh,
            pl.BlockSpec((1, 1), lambda i: (0, 0)),
        ],
        out_specs=[
            pl.BlockSpec((BN_, H), lambda i: (i, 0)),
            pl.BlockSpec((BN_, H), lambda i: (i, 0)),
        ],
        out_shape=[
            jax.ShapeDtypeStruct((N, H), jnp.float32),
            jax.ShapeDtypeStruct((N, H), jnp.float32),
        ],
        compiler_params=pltpu.CompilerParams(
            dimension_semantics=("parallel",)),
    )(x1h, a2h, w2a, b2a, g2, b2, wl1, bl1, wx1, bx1, wx2, bx2, lng, lnb, eps)


BQ = 512
BQC = 256


def _tc_cnmm_body(cn_ref, x3_ref, o_ref):
    o_ref[...] = jnp.dot(cn_ref[...], x3_ref[...],
                         preferred_element_type=jnp.float32)


def _tc_cnmm(cn, x3):
    grid = (Q // BQC,)
    return pl.pallas_call(
        _tc_cnmm_body,
        grid=grid,
        in_specs=[
            pl.BlockSpec((BQC, NPC), lambda i: (i, 0)),
            pl.BlockSpec((NPC, H), lambda i: (0, 0)),
        ],
        out_specs=pl.BlockSpec((BQC, H), lambda i: (i, 0)),
        out_shape=jax.ShapeDtypeStruct((Q, H), jnp.float32),
        compiler_params=pltpu.CompilerParams(
            dimension_semantics=("parallel",)),
    )(cn, x3)


def _tc_final_body(xcn_ref, xi_ref, xj_ref,
                   wi1_ref, bi1_ref, lnig_ref, lnib_ref, wi2_ref, bi2_ref,
                   wc1_ref, bc1_ref, wc2_ref, bc2_ref, lncg_ref, lncb_ref,
                   wc3_ref, bc3_ref, beta_ref,
                   wl1_ref, bl1_ref, ln1g_ref, ln1b_ref,
                   wl2_ref, bl2_ref, ln2g_ref, ln2b_ref,
                   wl3_ref, bl3_ref, o_ref):
    dot = lambda a, w, b: jnp.dot(a, w[...],
                                  preferred_element_type=jnp.float32) + b[...]
    hij = dot(xi_ref[...] * xj_ref[...], wi1_ref, bi1_ref)
    hij = jnp.maximum(_ln(hij, lnig_ref[...], lnib_ref[...]), 0.0)
    xij = dot(hij, wi2_ref, bi2_ref)
    hc = jnp.maximum(dot(xcn_ref[...], wc1_ref, bc1_ref), 0.0)
    hc = dot(hc, wc2_ref, bc2_ref)
    hc = jnp.maximum(_ln(hc, lncg_ref[...], lncb_ref[...]), 0.0)
    hc = dot(hc, wc3_ref, bc3_ref)
    pre = hc * beta_ref[0, 0] + xij
    o = dot(pre, wl1_ref, bl1_ref)
    o = jnp.maximum(_ln(o, ln1g_ref[...], ln1b_ref[...]), 0.0)
    o = dot(o, wl2_ref, bl2_ref)
    o = jnp.maximum(_ln(o, ln2g_ref[...], ln2b_ref[...]), 0.0)
    o_ref[...] = dot(o, wl3_ref, bl3_ref)


def _tc_final(xcn, xi, xj, args):
    grid = (Q // BQ,)
    hh = pl.BlockSpec((H, H), lambda i: (0, 0))
    vh = pl.BlockSpec((1, H), lambda i: (0, 0))
    qh = pl.BlockSpec((BQ, H), lambda i: (i, 0))
    return pl.pallas_call(
        _tc_final_body,
        grid=grid,
        in_specs=[
            qh, qh, qh,
            hh, vh, vh, vh, hh, vh,
            hh, vh, hh, vh, vh, vh, hh, vh,
            pl.BlockSpec((1, 1), lambda i: (0, 0)),
            hh, vh, vh, vh,
            hh, vh, vh, vh,
            pl.BlockSpec((H, D), lambda i: (0, 0)),
            pl.BlockSpec((1, D), lambda i: (0, 0)),
        ],
        out_specs=pl.BlockSpec((BQ, D), lambda i: (i, 0)),
        out_shape=jax.ShapeDtypeStruct((Q, D), jnp.float32),
        compiler_params=pltpu.CompilerParams(
            dimension_semantics=("parallel",)),
    )(xcn, xi, xj, *args)


def kernel(x, edge_index, adj, pos_edge, params):
    p = params
    zeros = jnp.zeros((RT_A, D), jnp.float32)
    r2 = lambda v: v.reshape(1, -1)

    echunks = jnp.pad(edge_index.reshape(2, E // KCH, KCH).transpose(1, 0, 2),
                      ((0, 8), (0, 0), (0, 0)))
    aggp = _sc_agg_edges(x, echunks, zeros).reshape(2, N, D)
    x1h = _tc_mlp1(x, aggp, p['W1a'], r2(p['b1a']), p['W1b'], r2(p['b1b']),
                   r2(p['bn1_g']), r2(p['bn1_b']),
                   p['eps1'].reshape(1, 1).astype(jnp.float32))
    x1flat = x1h.reshape(2 * N, D)
    a2h = _sc_agg_feat(x1flat, echunks, zeros).reshape(2, N, D)
    xl, x3 = _tc_mlp2(x1h, a2h, p['W2a'], r2(p['b2a']),
                      r2(p['bn2_g']), r2(p['bn2_b']),
                      p['Wl1'], r2(p['bl1']), p['Wx1'], r2(p['bx1']),
                      p['Wx2'], r2(p['bx2']), r2(p['lnx_g']), r2(p['lnx_b']),
                      p['eps2'].reshape(1, 1).astype(jnp.float32))

    tail = jnp.pad(adj[:, NMAIN:], ((0, 0), (0, WT - (N - NMAIN))))
    pairs = jnp.stack([pos_edge[0], pos_edge[1]], axis=1).ravel()
    posf = pos_edge.ravel()
    x3p = jnp.pad(x3, ((0, NPC - N), (0, 0)))
    cn, xi, xj = _sc_cn(adj, tail, pairs, posf, xl)
    xcn = _tc_cnmm(cn.reshape(Q, NPC), x3p)

    wl3 = jnp.pad(p['WL3'], ((0, 0), (0, D - p['WL3'].shape[1])))
    bl3 = jnp.pad(p['bL3'], (0, D - p['bL3'].shape[0])).reshape(1, D)
    args = (p['Wi1'], r2(p['bi1']), r2(p['lni_g']), r2(p['lni_b']),
            p['Wi2'], r2(p['bi2']),
            p['Wc1'], r2(p['bc1']), p['Wc2'], r2(p['bc2']),
            r2(p['lnc_g']), r2(p['lnc_b']), p['Wc3'], r2(p['bc3']),
            p['beta'].reshape(1, 1).astype(jnp.float32),
            p['WL1'], r2(p['bL1']), r2(p['lnL1_g']), r2(p['lnL1_b']),
            p['WL2'], r2(p['bL2']), r2(p['lnL2_g']), r2(p['lnL2_b']),
            wl3, bl3)
    o = _tc_final(xcn, xi, xj, args)
    return o[:, :7]


# speedup vs baseline: 1.0613x; 1.0004x over previous
import functools

import jax
import jax.numpy as jnp
import numpy as np
from jax import lax
from jax.experimental import pallas as pl
from jax.experimental.pallas import tpu as pltpu
from jax.experimental.pallas import tpu_sc as plsc

N = 10000
D = 128
H = 256
E = 320000
Q = 4096

NC = 2
NS = 16
KCH = 128
RT_A = 632
RT_LAST = N - (NS - 1) * RT_A

_BN_INV = 1.0 / np.sqrt(1.0 + 1e-5)


def _sc_mesh():
    return plsc.VectorSubcoreMesh(core_axis_name="c", subcore_axis_name="s",
                                  num_cores=NC, num_subcores=NS)


def _make_sc_agg(feature_split):
    ncht = E // KCH if feature_split else E // NC // KCH
    cpt = ncht // NS
    rem = ncht - cpt * NS
    cmax = cpt + (1 if rem else 0)

    @functools.partial(
        pl.kernel,
        out_type=jax.ShapeDtypeStruct((2 * N, D), jnp.float32),
        mesh=_sc_mesh(),
        scratch_types=[
            pltpu.VMEM((2, 1, 2, KCH), jnp.int32),
            pltpu.VMEM((2, KCH, D), jnp.float32),
            pltpu.VMEM_SHARED((N, D), jnp.float32),
            pltpu.SemaphoreType.DMA,
            pltpu.SemaphoreType.DMA,
            pltpu.SemaphoreType.DMA,
            pltpu.SemaphoreType.DMA,
        ],
    )
    def k(rows_hbm, echunks_hbm, zeros_hbm, out_hbm,
          eidx_v, rows_v, acc, sem0, sem1, ssem0, ssem1):
        sems = (sem0, sem1)
        ssems = (ssem0, ssem1)
        c = lax.axis_index("c")
        s = lax.axis_index("s")
        r0 = s * RT_A

        @pl.when(s < NS - 1)
        def _():
            pltpu.sync_copy(zeros_hbm, acc.at[pl.ds(r0, RT_A)])

        @pl.when(s == NS - 1)
        def _():
            pltpu.sync_copy(zeros_hbm.at[pl.ds(0, RT_LAST)],
                            acc.at[pl.ds((NS - 1) * RT_A, RT_LAST)])

        nmine = cpt + jnp.where(s < rem, 1, 0)
        cbase = s * cpt + jnp.minimum(s, rem)
        if not feature_split:
            cbase = cbase + c * ncht
        plsc.subcore_barrier()

        def fetch(i, b, scat_wait):
            w = pltpu.make_async_copy(rows_v.at[b], acc.at[pl.ds(0, KCH)],
                                      ssems[b])
            if scat_wait is None:
                w.wait()
            elif scat_wait is not False:
                @pl.when(scat_wait)
                def _():
                    w.wait()
            pltpu.sync_copy(echunks_hbm.at[pl.ds(cbase + i, 1)],
                            eidx_v.at[b])
            if feature_split:
                for j in range(KCH // 16):
                    sl = pl.ds(j * 16, 16)
                    eidx_v[b, 0, 0, sl] = eidx_v[b, 0, 0, sl] + c * N
            pltpu.async_copy(rows_hbm.at[eidx_v.at[b, 0, 0]], rows_v.at[b],
                             sems[b])

        def drain_scatter(b):
            pltpu.make_async_copy(rows_hbm.at[pl.ds(0, KCH)],
                                  rows_v.at[b], sems[b]).wait()
            pltpu.async_copy(rows_v.at[b], acc.at[eidx_v.at[b, 0, 1]],
                             ssems[b], add=True)

        fetch(0, 0, False)

        def body(k2, carry):
            i0 = 2 * k2
            fetch(i0 + 1, 1, k2 > 0)
            drain_scatter(0)

            @pl.when(i0 + 2 < nmine)
            def _():
                fetch(i0 + 2, 0, None)

            drain_scatter(1)
            return carry

        lax.fori_loop(0, cpt // 2, body, 0)

        @pl.when(nmine > cpt)
        def _():
            drain_scatter(0)

        for b in range(2):
            pltpu.make_async_copy(rows_v.at[b], acc.at[pl.ds(0, KCH)],
                                  ssems[b]).wait()

        plsc.subcore_barrier()

        @pl.when(s < NS - 1)
        def _():
            pltpu.sync_copy(acc.at[pl.ds(r0, RT_A)],
                            out_hbm.at[pl.ds(c * N + r0, RT_A)])

        @pl.when(s == NS - 1)
        def _():
            pltpu.sync_copy(acc.at[pl.ds((NS - 1) * RT_A, RT_LAST)],
                            out_hbm.at[pl.ds(c * N + (NS - 1) * RT_A, RT_LAST)])

    return k


_sc_agg_edges = _make_sc_agg(False)
_sc_agg_feat = _make_sc_agg(True)

QPT = Q // (NC * NS)
_CN_GRP = 4
_XCH = 16

W0 = 3328
WT = 128
NMAIN = 3 * W0
NPC = NMAIN + WT


def _make_sc_cn():
    qpt = Q // (NC * NS)
    ngrp = qpt // 4

    @functools.partial(
        pl.kernel,
        out_type=[
            jax.ShapeDtypeStruct((Q * NPC,), jnp.float32),
            jax.ShapeDtypeStruct((Q, H), jnp.float32),
            jax.ShapeDtypeStruct((Q, H), jnp.float32),
        ],
        mesh=_sc_mesh(),
        scratch_types=[
            pltpu.VMEM((_XCH,), jnp.int32),
            pltpu.VMEM((_XCH, H), jnp.float32),
            pltpu.VMEM((2, 8), jnp.int32),
            pltpu.VMEM((8, W0), jnp.float32),
            pltpu.VMEM((8, W0), jnp.float32),
            pltpu.VMEM((8, W0), jnp.float32),
            pltpu.VMEM((8, WT), jnp.float32),
            pltpu.VMEM((2, 4 * W0), jnp.float32),
            pltpu.VMEM((4 * WT,), jnp.float32),
            pltpu.SemaphoreType.DMA,
            pltpu.SemaphoreType.DMA,
            pltpu.SemaphoreType.DMA,
            pltpu.SemaphoreType.DMA,
            pltpu.SemaphoreType.DMA,
            pltpu.SemaphoreType.DMA,
            pltpu.SemaphoreType.DMA,
        ],
    )
    def _sc_cn(adj_hbm, tail_hbm, pairs_hbm, posf_hbm, xl_hbm,
               cn_hbm, xi_hbm, xj_hbm,
               idxx_v, xrows_v, pidx_v, bufA, bufB, bufC, bufL,
               st0_v, stL_v,
               semA, semB, semC, semL, wsem0a, wsem0b, semx):
        wsem0 = (wsem0a, wsem0b)
        c = lax.axis_index("c")
        s = lax.axis_index("s")
        wid = s * NC + c
        q0 = wid * qpt
        pbase = q0 * 2

        for t in range(qpt // _XCH):
            qt = q0 + t * _XCH
            pltpu.sync_copy(posf_hbm.at[pl.ds(qt, _XCH)], idxx_v)
            pltpu.async_copy(xl_hbm.at[idxx_v], xrows_v, semx).wait()
            pltpu.sync_copy(xrows_v, xi_hbm.at[pl.ds(qt, _XCH)])
            pltpu.sync_copy(posf_hbm.at[pl.ds(Q + qt, _XCH)], idxx_v)
            pltpu.async_copy(xl_hbm.at[idxx_v], xrows_v, semx).wait()
            pltpu.sync_copy(xrows_v, xj_hbm.at[pl.ds(qt, _XCH)])

        def load_pidx(g, iset):
            pltpu.sync_copy(pairs_hbm.at[pl.ds(pbase + g * 8, 8)],
                            pidx_v.at[iset])

        def fetch_w(iset, buf, sem, coff, w):
            pltpu.async_copy(adj_hbm.at[pidx_v.at[iset], pl.ds(coff, w)],
                             buf, sem)

        def fetch_tail(iset):
            pltpu.async_copy(tail_hbm.at[pidx_v.at[iset]], bufL, semL)

        def products(buf, store):
            w = buf.shape[1]

            def cols(cc, c2):
                for u in range(4):
                    for gg in range(4):
                        co = (cc * 4 + gg) * 16
                        sl = pl.ds(co, 16)
                        store(u, co, buf[2 * u, sl] * buf[2 * u + 1, sl])
                return c2

            lax.fori_loop(0, w // 64, cols, 0)

        def compute_w0(buf, set_, qb, coff, skip_wait):
            wait = pltpu.make_async_copy(
                st0_v.at[set_], cn_hbm.at[pl.ds(0, 4 * W0)], wsem0[set_])
            if skip_wait is None:
                wait.wait()
            else:
                @pl.when(skip_wait)
                def _():
                    wait.wait()
            products(buf, lambda u, co, v: st0_v.__setitem__(
                (set_, pl.ds(u * W0 + co, 16)), v))
            for u in range(4):
                pltpu.async_copy(
                    st0_v.at[set_, pl.ds(u * W0, W0)],
                    cn_hbm.at[pl.ds((qb + u) * NPC + coff, W0)],
                    wsem0[set_])

        def compute_sync(buf, st, qb, coff, w):
            products(buf, lambda u, co, v: st.__setitem__(
                (pl.ds(u * w + co, 16),), v))
            for u in range(4):
                pltpu.sync_copy(st.at[pl.ds(u * w, w)],
                                cn_hbm.at[pl.ds((qb + u) * NPC + coff, w)])

        def drain(buf, sem, coff, w):
            pltpu.make_async_copy(adj_hbm.at[pl.ds(0, 8), pl.ds(coff, w)],
                                  buf, sem).wait()

        def drain_tail():
            pltpu.make_async_copy(tail_hbm.at[pl.ds(0, 8)], bufL, semL).wait()

        load_pidx(0, 0)
        fetch_w(0, bufA, semA, 0, W0)
        fetch_w(0, bufB, semB, W0, W0)

        def section(g2, g, iset, so, first, last):
            qb = q0 + g * 4
            fetch_w(iset, bufC, semC, 2 * W0, W0)
            drain(bufA, semA, 0, W0)
            compute_w0(bufA, so, qb, 0, (g2 > 0) if first else None)
            fetch_tail(iset)
            drain(bufB, semB, W0, W0)
            compute_w0(bufB, 1 - so, qb, W0, (g2 > 0) if first else None)

            nxt = 1 - iset
            if last:
                @pl.when(g2 < ngrp // 2 - 1)
                def _():
                    load_pidx(g + 1, nxt)
                    fetch_w(nxt, bufA, semA, 0, W0)
            else:
                load_pidx(g + 1, nxt)
                fetch_w(nxt, bufA, semA, 0, W0)
            drain(bufC, semC, 2 * W0, W0)
            compute_w0(bufC, so, qb, 2 * W0, None)

            if last:
                @pl.when(g2 < ngrp // 2 - 1)
                def _():
                    fetch_w(nxt, bufB, semB, W0, W0)
            else:
                fetch_w(nxt, bufB, semB, W0, W0)
            drain_tail()
            compute_sync(bufL, stL_v, qb, NMAIN, WT)
            return qb

        def body(g2, carry):
            g = 2 * g2
            section(g2, g, 0, 0, True, False)
            section(g2, g + 1, 1, 1, False, True)
            return carry

        lax.fori_loop(0, ngrp // 2, body, 0)
        for set_ in range(2):
            pltpu.make_async_copy(st0_v.at[set_],
                                  cn_hbm.at[pl.ds(0, 4 * W0)],
                                  wsem0[set_]).wait()

    return _sc_cn


_sc_cn = _make_sc_cn()


def _ln(h, g, b):
    m = jnp.mean(h, axis=-1, keepdims=True)
    v = jnp.mean((h - m) ** 2, axis=-1, keepdims=True)
    return (h - m) * jax.lax.rsqrt(v + 1e-5) * g + b


BN_ = 1000


def _tc_mlp1_body(x_ref, aggp_ref, w1a_ref, b1a_ref, w1b_ref, b1b_ref,
                  g_ref, bb_ref, eps_ref, out_ref):
    h = x_ref[...] * (1.0 + eps_ref[0, 0]) + aggp_ref[0] + aggp_ref[1]
    h = jnp.maximum(jnp.dot(h, w1a_ref[...],
                            preferred_element_type=jnp.float32) + b1a_ref[...], 0.0)
    h = jnp.maximum(jnp.dot(h, w1b_ref[...],
                            preferred_element_type=jnp.float32) + b1b_ref[...], 0.0)
    y = h * (_BN_INV * g_ref[...]) + bb_ref[...]
    out_ref[0] = y[:, :D]
    out_ref[1] = y[:, D:]


def _tc_mlp1(x, aggp, w1a, b1a, w1b, b1b, g, b, eps):
    grid = (N // BN_,)
    return pl.pallas_call(
        _tc_mlp1_body,
        grid=grid,
        in_specs=[
            pl.BlockSpec((BN_, D), lambda i: (i, 0)),
            pl.BlockSpec((2, BN_, D), lambda i: (0, i, 0)),
            pl.BlockSpec((D, H), lambda i: (0, 0)),
            pl.BlockSpec((1, H), lambda i: (0, 0)),
            pl.BlockSpec((H, H), lambda i: (0, 0)),
            pl.BlockSpec((1, H), lambda i: (0, 0)),
            pl.BlockSpec((1, H), lambda i: (0, 0)),
            pl.BlockSpec((1, H), lambda i: (0, 0)),
            pl.BlockSpec((1, 1), lambda i: (0, 0)),
        ],
        out_specs=pl.BlockSpec((2, BN_, D), lambda i: (0, i, 0)),
        out_shape=jax.ShapeDtypeStruct((2, N, D), jnp.float32),
        compiler_params=pltpu.CompilerParams(
            dimension_semantics=("parallel",)),
    )(x, aggp, w1a, b1a, w1b, b1b, g, b, eps)


def _tc_mlp2_body(x1h_ref, a2h_ref, w2a_ref, b2a_ref, g2_ref, bb2_ref,
                  wl1_ref, bl1_ref, wx1_ref, bx1_ref, wx2_ref, bx2_ref,
                  lng_ref, lnb_ref, eps_ref, xl_ref, x3_ref):
    e = 1.0 + eps_ref[0, 0]
    ta = x1h_ref[0] * e + a2h_ref[0]
    tb = x1h_ref[1] * e + a2h_ref[1]
    h = (jnp.dot(ta, w2a_ref[:D, :], preferred_element_type=jnp.float32)
         + jnp.dot(tb, w2a_ref[D:, :], preferred_element_type=jnp.float32)
         + b2a_ref[...])
    h = jnp.maximum(h, 0.0)
    x2 = h * (_BN_INV * g2_ref[...]) + bb2_ref[...]
    xl = jnp.dot(x2, wl1_ref[...], preferred_element_type=jnp.float32) + bl1_ref[...]
    hx = jnp.maximum(jnp.dot(xl, wx1_ref[...],
                             preferred_element_type=jnp.float32) + bx1_ref[...], 0.0)
    hx = jnp.dot(hx, wx2_ref[...], preferred_element_type=jnp.float32) + bx2_ref[...]
    hx = jnp.maximum(_ln(hx, lng_ref[...], lnb_ref[...]), 0.0)
    xl_ref[...] = xl
    x3_ref[...] = xl + hx


def _tc_mlp2(x1h, a2h, w2a, b2a, g2, b2, wl1, bl1, wx1, bx1, wx2, bx2,
             lng, lnb, eps):
    grid = (N // BN_,)
    hh = pl.BlockSpec((H, H), lambda i: (0, 0))
    vh = pl.BlockSpec((1, H), lambda i: (0, 0))
    return pl.pallas_call(
        _tc_mlp2_body,
        grid=grid,
        in_specs=[
            pl.BlockSpec((2, BN_, D), lambda i: (0, i, 0)),
            pl.BlockSpec((2, BN_, D), lambda i: (0, i, 0)),
            hh, vh, vh, vh,
            hh, vh, hh, vh, hh, vh,
            vh, vh,
            pl.BlockSpec((1, 1), lambda i: (0, 0)),
        ],
        out_specs=[
            pl.BlockSpec((BN_, H), lambda i: (i, 0)),
            pl.BlockSpec((BN_, H), lambda i: (i, 0)),
        ],
        out_shape=[
            jax.ShapeDtypeStruct((N, H), jnp.float32),
            jax.ShapeDtypeStruct((N, H), jnp.float32),
        ],
        compiler_params=pltpu.CompilerParams(
            dimension_semantics=("parallel",)),
    )(x1h, a2h, w2a, b2a, g2, b2, wl1, bl1, wx1, bx1, wx2, bx2, lng, lnb, eps)


BQ = 512
BQC = 256


def _tc_cnmm_body(cn_ref, x3_ref, o_ref):
    o_ref[...] = jnp.dot(cn_ref[...], x3_ref[...],
                         preferred_element_type=jnp.float32)


def _tc_cnmm(cn, x3):
    grid = (Q // BQC,)
    return pl.pallas_call(
        _tc_cnmm_body,
        grid=grid,
        in_specs=[
            pl.BlockSpec((BQC, NPC), lambda i: (i, 0)),
            pl.BlockSpec((NPC, H), lambda i: (0, 0)),
        ],
        out_specs=pl.BlockSpec((BQC, H), lambda i: (i, 0)),
        out_shape=jax.ShapeDtypeStruct((Q, H), jnp.float32),
        compiler_params=pltpu.CompilerParams(
            dimension_semantics=("parallel",)),
    )(cn, x3)


def _tc_final_body(xcn_ref, xi_ref, xj_ref,
                   wi1_ref, bi1_ref, lnig_ref, lnib_ref, wi2_ref, bi2_ref,
                   wc1_ref, bc1_ref, wc2_ref, bc2_ref, lncg_ref, lncb_ref,
                   wc3_ref, bc3_ref, beta_ref,
                   wl1_ref, bl1_ref, ln1g_ref, ln1b_ref,
                   wl2_ref, bl2_ref, ln2g_ref, ln2b_ref,
                   wl3_ref, bl3_ref, o_ref):
    dot = lambda a, w, b: jnp.dot(a, w[...],
                                  preferred_element_type=jnp.float32) + b[...]
    hij = dot(xi_ref[...] * xj_ref[...], wi1_ref, bi1_ref)
    hij = jnp.maximum(_ln(hij, lnig_ref[...], lnib_ref[...]), 0.0)
    xij = dot(hij, wi2_ref, bi2_ref)
    hc = jnp.maximum(dot(xcn_ref[...], wc1_ref, bc1_ref), 0.0)
    hc = dot(hc, wc2_ref, bc2_ref)
    hc = jnp.maximum(_ln(hc, lncg_ref[...], lncb_ref[...]), 0.0)
    hc = dot(hc, wc3_ref, bc3_ref)
    pre = hc * beta_ref[0, 0] + xij
    o = dot(pre, wl1_ref, bl1_ref)
    o = jnp.maximum(_ln(o, ln1g_ref[...], ln1b_ref[...]), 0.0)
    o = dot(o, wl2_ref, bl2_ref)
    o = jnp.maximum(_ln(o, ln2g_ref[...], ln2b_ref[...]), 0.0)
    o_ref[...] = dot(o, wl3_ref, bl3_ref)


def _tc_final(xcn, xi, xj, args):
    grid = (Q // BQ,)
    hh = pl.BlockSpec((H, H), lambda i: (0, 0))
    vh = pl.BlockSpec((1, H), lambda i: (0, 0))
    qh = pl.BlockSpec((BQ, H), lambda i: (i, 0))
    return pl.pallas_call(
        _tc_final_body,
        grid=grid,
        in_specs=[
            qh, qh, qh,
            hh, vh, vh, vh, hh, vh,
            hh, vh, hh, vh, vh, vh, hh, vh,
            pl.BlockSpec((1, 1), lambda i: (0, 0)),
            hh, vh, vh, vh,
            hh, vh, vh, vh,
            pl.BlockSpec((H, D), lambda i: (0, 0)),
            pl.BlockSpec((1, D), lambda i: (0, 0)),
        ],
        out_specs=pl.BlockSpec((BQ, D), lambda i: (i, 0)),
        out_shape=jax.ShapeDtypeStruct((Q, D), jnp.float32),
        compiler_params=pltpu.CompilerParams(
            dimension_semantics=("parallel",)),
    )(xcn, xi, xj, *args)


def kernel(x, edge_index, adj, pos_edge, params):
    p = params
    zeros = jnp.zeros((RT_A, D), jnp.float32)
    r2 = lambda v: v.reshape(1, -1)

    echunks = jnp.pad(edge_index.reshape(2, E // KCH, KCH).transpose(1, 0, 2),
                      ((0, 8), (0, 0), (0, 0)))
    aggp = _sc_agg_edges(x, echunks, zeros).reshape(2, N, D)
    x1h = _tc_mlp1(x, aggp, p['W1a'], r2(p['b1a']), p['W1b'], r2(p['b1b']),
                   r2(p['bn1_g']), r2(p['bn1_b']),
                   p['eps1'].reshape(1, 1).astype(jnp.float32))
    x1flat = x1h.reshape(2 * N, D)
    a2h = _sc_agg_feat(x1flat, echunks, zeros).reshape(2, N, D)
    xl, x3 = _tc_mlp2(x1h, a2h, p['W2a'], r2(p['b2a']),
                      r2(p['bn2_g']), r2(p['bn2_b']),
                      p['Wl1'], r2(p['bl1']), p['Wx1'], r2(p['bx1']),
                      p['Wx2'], r2(p['bx2']), r2(p['lnx_g']), r2(p['lnx_b']),
                      p['eps2'].reshape(1, 1).astype(jnp.float32))

    tail = jnp.pad(adj[:, NMAIN:], ((0, 0), (0, WT - (N - NMAIN))))
    pairs = jnp.stack([pos_edge[0], pos_edge[1]], axis=1).ravel()
    posf = pos_edge.ravel()
    x3p = jnp.pad(x3, ((0, NPC - N), (0, 0)))
    cn, xi, xj = _sc_cn(adj, tail, pairs, posf, xl)
    xcn = _tc_cnmm(cn.reshape(Q, NPC), x3p)

    wl3 = jnp.pad(p['WL3'], ((0, 0), (0, D - p['WL3'].shape[1])))
    bl3 = jnp.pad(p['bL3'], (0, D - p['bL3'].shape[0])).reshape(1, D)
    args = (p['Wi1'], r2(p['bi1']), r2(p['lni_g']), r2(p['lni_b']),
            p['Wi2'], r2(p['bi2']),
            p['Wc1'], r2(p['bc1']), p['Wc2'], r2(p['bc2']),
            r2(p['lnc_g']), r2(p['lnc_b']), p['Wc3'], r2(p['bc3']),
            p['beta'].reshape(1, 1).astype(jnp.float32),
            p['WL1'], r2(p['bL1']), r2(p['lnL1_g']), r2(p['lnL1_b']),
            p['WL2'], r2(p['bL2']), r2(p['lnL2_g']), r2(p['lnL2_b']),
            wl3, bl3)
    o = _tc_final(xcn, xi, xj, args)
    return o[:, :7]


# speedup vs baseline: 1.0702x; 1.0083x over previous
import functools

import jax
import jax.numpy as jnp
import numpy as np
from jax import lax
from jax.experimental import pallas as pl
from jax.experimental.pallas import tpu as pltpu
from jax.experimental.pallas import tpu_sc as plsc

N = 10000
D = 128
H = 256
E = 320000
Q = 4096

NC = 2
NS = 16
KCH = 128
RT_A = 632
RT_LAST = N - (NS - 1) * RT_A

_BN_INV = 1.0 / np.sqrt(1.0 + 1e-5)


def _sc_mesh():
    return plsc.VectorSubcoreMesh(core_axis_name="c", subcore_axis_name="s",
                                  num_cores=NC, num_subcores=NS)


def _make_sc_agg(feature_split):
    ncht = E // KCH if feature_split else E // NC // KCH
    cpt = ncht // NS
    rem = ncht - cpt * NS
    cmax = cpt + (1 if rem else 0)

    @functools.partial(
        pl.kernel,
        out_type=jax.ShapeDtypeStruct((2 * N, D), jnp.float32),
        mesh=_sc_mesh(),
        scratch_types=[
            pltpu.VMEM((2, 1, 2, KCH), jnp.int32),
            pltpu.VMEM((2, KCH, D), jnp.float32),
            pltpu.VMEM_SHARED((N, D), jnp.float32),
            pltpu.SemaphoreType.DMA,
            pltpu.SemaphoreType.DMA,
            pltpu.SemaphoreType.DMA,
            pltpu.SemaphoreType.DMA,
        ],
    )
    def k(rows_hbm, echunks_hbm, zeros_hbm, out_hbm,
          eidx_v, rows_v, acc, sem0, sem1, ssem0, ssem1):
        sems = (sem0, sem1)
        ssems = (ssem0, ssem1)
        c = lax.axis_index("c")
        s = lax.axis_index("s")
        r0 = s * RT_A

        @pl.when(s < NS - 1)
        def _():
            pltpu.sync_copy(zeros_hbm, acc.at[pl.ds(r0, RT_A)])

        @pl.when(s == NS - 1)
        def _():
            pltpu.sync_copy(zeros_hbm.at[pl.ds(0, RT_LAST)],
                            acc.at[pl.ds((NS - 1) * RT_A, RT_LAST)])

        nmine = cpt + jnp.where(s < rem, 1, 0)
        cbase = s * cpt + jnp.minimum(s, rem)
        if not feature_split:
            cbase = cbase + c * ncht
        plsc.subcore_barrier()

        def fetch(i, b, scat_wait):
            w = pltpu.make_async_copy(rows_v.at[b], acc.at[pl.ds(0, KCH)],
                                      ssems[b])
            if scat_wait is None:
                w.wait()
            elif scat_wait is not False:
                @pl.when(scat_wait)
                def _():
                    w.wait()
            pltpu.sync_copy(echunks_hbm.at[pl.ds(cbase + i, 1)],
                            eidx_v.at[b])
            if feature_split:
                for j in range(KCH // 16):
                    sl = pl.ds(j * 16, 16)
                    eidx_v[b, 0, 0, sl] = eidx_v[b, 0, 0, sl] + c * N
            pltpu.async_copy(rows_hbm.at[eidx_v.at[b, 0, 0]], rows_v.at[b],
                             sems[b])

        def drain_scatter(b):
            pltpu.make_async_copy(rows_hbm.at[pl.ds(0, KCH)],
                                  rows_v.at[b], sems[b]).wait()
            pltpu.async_copy(rows_v.at[b], acc.at[eidx_v.at[b, 0, 1]],
                             ssems[b], add=True)

        fetch(0, 0, False)

        def body(k2, carry):
            i0 = 2 * k2
            fetch(i0 + 1, 1, k2 > 0)
            drain_scatter(0)

            @pl.when(i0 + 2 < nmine)
            def _():
                fetch(i0 + 2, 0, None)

            drain_scatter(1)
            return carry

        lax.fori_loop(0, cpt // 2, body, 0)

        @pl.when(nmine > cpt)
        def _():
            drain_scatter(0)

        for b in range(2):
            pltpu.make_async_copy(rows_v.at[b], acc.at[pl.ds(0, KCH)],
                                  ssems[b]).wait()

        plsc.subcore_barrier()

        @pl.when(s < NS - 1)
        def _():
            pltpu.sync_copy(acc.at[pl.ds(r0, RT_A)],
                            out_hbm.at[pl.ds(c * N + r0, RT_A)])

        @pl.when(s == NS - 1)
        def _():
            pltpu.sync_copy(acc.at[pl.ds((NS - 1) * RT_A, RT_LAST)],
                            out_hbm.at[pl.ds(c * N + (NS - 1) * RT_A, RT_LAST)])

    return k


_sc_agg_edges = _make_sc_agg(False)
_sc_agg_feat = _make_sc_agg(True)

QPT = Q // (NC * NS)
_CN_GRP = 4
_XCH = 16

W0 = 3328
WT = 128
NMAIN = 3 * W0
NPC = NMAIN + WT


def _make_sc_cn():
    qpt = Q // (NC * NS)
    ngrp = qpt // 4

    @functools.partial(
        pl.kernel,
        out_type=[
            jax.ShapeDtypeStruct((Q * NPC,), jnp.float32),
            jax.ShapeDtypeStruct((Q, H), jnp.float32),
            jax.ShapeDtypeStruct((Q, H), jnp.float32),
        ],
        mesh=_sc_mesh(),
        scratch_types=[
            pltpu.VMEM((_XCH,), jnp.int32),
            pltpu.VMEM((_XCH, H), jnp.float32),
            pltpu.VMEM((2, 8), jnp.int32),
            pltpu.VMEM((8, W0), jnp.float32),
            pltpu.VMEM((8, W0), jnp.float32),
            pltpu.VMEM((8, W0), jnp.float32),
            pltpu.VMEM((8, WT), jnp.float32),
            pltpu.VMEM((4 * NPC,), jnp.float32),
            pltpu.SemaphoreType.DMA,
            pltpu.SemaphoreType.DMA,
            pltpu.SemaphoreType.DMA,
            pltpu.SemaphoreType.DMA,
            pltpu.SemaphoreType.DMA,
            pltpu.SemaphoreType.DMA,
        ],
    )
    def _sc_cn(adj_hbm, tail_hbm, pairs_hbm, posf_hbm, xl_hbm,
               cn_hbm, xi_hbm, xj_hbm,
               idxx_v, xrows_v, pidx_v, bufA, bufB, bufC, bufL,
               st_v,
               semA, semB, semC, semL, wsem, semx):
        c = lax.axis_index("c")
        s = lax.axis_index("s")
        wid = s * NC + c
        q0 = wid * qpt
        pbase = q0 * 2

        for t in range(qpt // _XCH):
            qt = q0 + t * _XCH
            pltpu.sync_copy(posf_hbm.at[pl.ds(qt, _XCH)], idxx_v)
            pltpu.async_copy(xl_hbm.at[idxx_v], xrows_v, semx).wait()
            pltpu.sync_copy(xrows_v, xi_hbm.at[pl.ds(qt, _XCH)])
            pltpu.sync_copy(posf_hbm.at[pl.ds(Q + qt, _XCH)], idxx_v)
            pltpu.async_copy(xl_hbm.at[idxx_v], xrows_v, semx).wait()
            pltpu.sync_copy(xrows_v, xj_hbm.at[pl.ds(qt, _XCH)])

        def load_pidx(g, iset):
            pltpu.sync_copy(pairs_hbm.at[pl.ds(pbase + g * 8, 8)],
                            pidx_v.at[iset])

        def fetch_w(iset, buf, sem, coff, w):
            pltpu.async_copy(adj_hbm.at[pidx_v.at[iset], pl.ds(coff, w)],
                             buf, sem)

        def fetch_tail(iset):
            pltpu.async_copy(tail_hbm.at[pidx_v.at[iset]], bufL, semL)

        def products(buf, store):
            w = buf.shape[1]

            def cols(cc, c2):
                for u in range(4):
                    for gg in range(4):
                        co = (cc * 4 + gg) * 16
                        sl = pl.ds(co, 16)
                        store(u, co, buf[2 * u, sl] * buf[2 * u + 1, sl])
                return c2

            lax.fori_loop(0, w // 64, cols, 0)

        def wait_writes(skip_wait):
            wait = pltpu.make_async_copy(
                st_v, cn_hbm.at[pl.ds(0, 4 * NPC)], wsem)
            if skip_wait is None:
                wait.wait()
            else:
                @pl.when(skip_wait)
                def _():
                    wait.wait()

        def compute_win(buf, coff):
            products(buf, lambda u, co, v: st_v.__setitem__(
                (pl.ds(u * NPC + coff + co, 16),), v))

        def write_rows(qb):
            for u in range(4):
                pltpu.async_copy(st_v.at[pl.ds(u * NPC, NPC)],
                                 cn_hbm.at[pl.ds((qb + u) * NPC, NPC)],
                                 wsem)

        def drain(buf, sem, coff, w):
            pltpu.make_async_copy(adj_hbm.at[pl.ds(0, 8), pl.ds(coff, w)],
                                  buf, sem).wait()

        def drain_tail():
            pltpu.make_async_copy(tail_hbm.at[pl.ds(0, 8)], bufL, semL).wait()

        load_pidx(0, 0)
        fetch_w(0, bufA, semA, 0, W0)
        fetch_w(0, bufB, semB, W0, W0)

        def section(g2, g, iset, skip_wait, last):
            qb = q0 + g * 4
            fetch_w(iset, bufC, semC, 2 * W0, W0)
            drain(bufA, semA, 0, W0)
            wait_writes(skip_wait)
            compute_win(bufA, 0)
            fetch_tail(iset)
            drain(bufB, semB, W0, W0)
            compute_win(bufB, W0)

            nxt = 1 - iset
            if last:
                @pl.when(g2 < ngrp // 2 - 1)
                def _():
                    load_pidx(g + 1, nxt)
                    fetch_w(nxt, bufA, semA, 0, W0)
            else:
                load_pidx(g + 1, nxt)
                fetch_w(nxt, bufA, semA, 0, W0)
            drain(bufC, semC, 2 * W0, W0)
            compute_win(bufC, 2 * W0)

            if last:
                @pl.when(g2 < ngrp // 2 - 1)
                def _():
                    fetch_w(nxt, bufB, semB, W0, W0)
            else:
                fetch_w(nxt, bufB, semB, W0, W0)
            drain_tail()
            compute_win(bufL, NMAIN)
            write_rows(qb)

        def body(g2, carry):
            g = 2 * g2
            section(g2, g, 0, g2 > 0, False)
            section(g2, g + 1, 1, None, True)
            return carry

        lax.fori_loop(0, ngrp // 2, body, 0)
        pltpu.make_async_copy(st_v, cn_hbm.at[pl.ds(0, 4 * NPC)],
                              wsem).wait()

    return _sc_cn


_sc_cn = _make_sc_cn()


def _ln(h, g, b):
    m = jnp.mean(h, axis=-1, keepdims=True)
    v = jnp.mean((h - m) ** 2, axis=-1, keepdims=True)
    return (h - m) * jax.lax.rsqrt(v + 1e-5) * g + b


BN_ = 1000


def _tc_mlp1_body(x_ref, aggp_ref, w1a_ref, b1a_ref, w1b_ref, b1b_ref,
                  g_ref, bb_ref, eps_ref, out_ref):
    h = x_ref[...] * (1.0 + eps_ref[0, 0]) + aggp_ref[0] + aggp_ref[1]
    h = jnp.maximum(jnp.dot(h, w1a_ref[...],
                            preferred_element_type=jnp.float32) + b1a_ref[...], 0.0)
    h = jnp.maximum(jnp.dot(h, w1b_ref[...],
                            preferred_element_type=jnp.float32) + b1b_ref[...], 0.0)
    y = h * (_BN_INV * g_ref[...]) + bb_ref[...]
    out_ref[0] = y[:, :D]
    out_ref[1] = y[:, D:]


def _tc_mlp1(x, aggp, w1a, b1a, w1b, b1b, g, b, eps):
    grid = (N // BN_,)
    return pl.pallas_call(
        _tc_mlp1_body,
        grid=grid,
        in_specs=[
            pl.BlockSpec((BN_, D), lambda i: (i, 0)),
            pl.BlockSpec((2, BN_, D), lambda i: (0, i, 0)),
            pl.BlockSpec((D, H), lambda i: (0, 0)),
            pl.BlockSpec((1, H), lambda i: (0, 0)),
            pl.BlockSpec((H, H), lambda i: (0, 0)),
            pl.BlockSpec((1, H), lambda i: (0, 0)),
            pl.BlockSpec((1, H), lambda i: (0, 0)),
            pl.BlockSpec((1, H), lambda i: (0, 0)),
            pl.BlockSpec((1, 1), lambda i: (0, 0)),
        ],
        out_specs=pl.BlockSpec((2, BN_, D), lambda i: (0, i, 0)),
        out_shape=jax.ShapeDtypeStruct((2, N, D), jnp.float32),
        compiler_params=pltpu.CompilerParams(
            dimension_semantics=("parallel",)),
    )(x, aggp, w1a, b1a, w1b, b1b, g, b, eps)


def _tc_mlp2_body(x1h_ref, a2h_ref, w2a_ref, b2a_ref, g2_ref, bb2_ref,
                  wl1_ref, bl1_ref, wx1_ref, bx1_ref, wx2_ref, bx2_ref,
                  lng_ref, lnb_ref, eps_ref, xl_ref, x3_ref):
    e = 1.0 + eps_ref[0, 0]
    ta = x1h_ref[0] * e + a2h_ref[0]
    tb = x1h_ref[1] * e + a2h_ref[1]
    h = (jnp.dot(ta, w2a_ref[:D, :], preferred_element_type=jnp.float32)
         + jnp.dot(tb, w2a_ref[D:, :], preferred_element_type=jnp.float32)
         + b2a_ref[...])
    h = jnp.maximum(h, 0.0)
    x2 = h * (_BN_INV * g2_ref[...]) + bb2_ref[...]
    xl = jnp.dot(x2, wl1_ref[...], preferred_element_type=jnp.float32) + bl1_ref[...]
    hx = jnp.maximum(jnp.dot(xl, wx1_ref[...],
                             preferred_element_type=jnp.float32) + bx1_ref[...], 0.0)
    hx = jnp.dot(hx, wx2_ref[...], preferred_element_type=jnp.float32) + bx2_ref[...]
    hx = jnp.maximum(_ln(hx, lng_ref[...], lnb_ref[...]), 0.0)
    xl_ref[...] = xl
    x3_ref[...] = xl + hx


def _tc_mlp2(x1h, a2h, w2a, b2a, g2, b2, wl1, bl1, wx1, bx1, wx2, bx2,
             lng, lnb, eps):
    grid = (N // BN_,)
    hh = pl.BlockSpec((H, H), lambda i: (0, 0))
    vh = pl.BlockSpec((1, H), lambda i: (0, 0))
    return pl.pallas_call(
        _tc_mlp2_body,
        grid=grid,
        in_specs=[
            pl.BlockSpec((2, BN_, D), lambda i: (0, i, 0)),
            pl.BlockSpec((2, BN_, D), lambda i: (0, i, 0)),
            hh, vh, vh, vh,
            hh, vh, hh, vh, hh, vh,
            vh, vh,
            pl.BlockSpec((1, 1), lambda i: (0, 0)),
        ],
        out_specs=[
            pl.BlockSpec((BN_, H), lambda i: (i, 0)),
            pl.BlockSpec((BN_, H), lambda i: (i, 0)),
        ],
        out_shape=[
            jax.ShapeDtypeStruct((N, H), jnp.float32),
            jax.ShapeDtypeStruct((N, H), jnp.float32),
        ],
        compiler_params=pltpu.CompilerParams(
            dimension_semantics=("parallel",)),
    )(x1h, a2h, w2a, b2a, g2, b2, wl1, bl1, wx1, bx1, wx2, bx2, lng, lnb, eps)


BQ = 512
BQC = 256


def _tc_cnmm_body(cn_ref, x3_ref, o_ref):
    o_ref[...] = jnp.dot(cn_ref[...], x3_ref[...],
                         preferred_element_type=jnp.float32)


def _tc_cnmm(cn, x3):
    grid = (Q // BQC,)
    return pl.pallas_call(
        _tc_cnmm_body,
        grid=grid,
        in_specs=[
            pl.BlockSpec((BQC, NPC), lambda i: (i, 0)),
            pl.BlockSpec((NPC, H), lambda i: (0, 0)),
        ],
        out_specs=pl.BlockSpec((BQC, H), lambda i: (i, 0)),
        out_shape=jax.ShapeDtypeStruct((Q, H), jnp.float32),
        compiler_params=pltpu.CompilerParams(
            dimension_semantics=("parallel",)),
    )(cn, x3)


def _tc_final_body(xcn_ref, xi_ref, xj_ref,
                   wi1_ref, bi1_ref, lnig_ref, lnib_ref, wi2_ref, bi2_ref,
                   wc1_ref, bc1_ref, wc2_ref, bc2_ref, lncg_ref, lncb_ref,
                   wc3_ref, bc3_ref, beta_ref,
                   wl1_ref, bl1_ref, ln1g_ref, ln1b_ref,
                   wl2_ref, bl2_ref, ln2g_ref, ln2b_ref,
                   wl3_ref, bl3_ref, o_ref):
    dot = lambda a, w, b: jnp.dot(a, w[...],
                                  preferred_element_type=jnp.float32) + b[...]
    hij = dot(xi_ref[...] * xj_ref[...], wi1_ref, bi1_ref)
    hij = jnp.maximum(_ln(hij, lnig_ref[...], lnib_ref[...]), 0.0)
    xij = dot(hij, wi2_ref, bi2_ref)
    hc = jnp.maximum(dot(xcn_ref[...], wc1_ref, bc1_ref), 0.0)
    hc = dot(hc, wc2_ref, bc2_ref)
    hc = jnp.maximum(_ln(hc, lncg_ref[...], lncb_ref[...]), 0.0)
    hc = dot(hc, wc3_ref, bc3_ref)
    pre = hc * beta_ref[0, 0] + xij
    o = dot(pre, wl1_ref, bl1_ref)
    o = jnp.maximum(_ln(o, ln1g_ref[...], ln1b_ref[...]), 0.0)
    o = dot(o, wl2_ref, bl2_ref)
    o = jnp.maximum(_ln(o, ln2g_ref[...], ln2b_ref[...]), 0.0)
    o_ref[...] = dot(o, wl3_ref, bl3_ref)


def _tc_final(xcn, xi, xj, args):
    grid = (Q // BQ,)
    hh = pl.BlockSpec((H, H), lambda i: (0, 0))
    vh = pl.BlockSpec((1, H), lambda i: (0, 0))
    qh = pl.BlockSpec((BQ, H), lambda i: (i, 0))
    return pl.pallas_call(
        _tc_final_body,
        grid=grid,
        in_specs=[
            qh, qh, qh,
            hh, vh, vh, vh, hh, vh,
            hh, vh, hh, vh, vh, vh, hh, vh,
            pl.BlockSpec((1, 1), lambda i: (0, 0)),
            hh, vh, vh, vh,
            hh, vh, vh, vh,
            pl.BlockSpec((H, D), lambda i: (0, 0)),
            pl.BlockSpec((1, D), lambda i: (0, 0)),
        ],
        out_specs=pl.BlockSpec((BQ, D), lambda i: (i, 0)),
        out_shape=jax.ShapeDtypeStruct((Q, D), jnp.float32),
        compiler_params=pltpu.CompilerParams(
            dimension_semantics=("parallel",)),
    )(xcn, xi, xj, *args)


def kernel(x, edge_index, adj, pos_edge, params):
    p = params
    zeros = jnp.zeros((RT_A, D), jnp.float32)
    r2 = lambda v: v.reshape(1, -1)

    echunks = jnp.pad(edge_index.reshape(2, E // KCH, KCH).transpose(1, 0, 2),
                      ((0, 8), (0, 0), (0, 0)))
    aggp = _sc_agg_edges(x, echunks, zeros).reshape(2, N, D)
    x1h = _tc_mlp1(x, aggp, p['W1a'], r2(p['b1a']), p['W1b'], r2(p['b1b']),
                   r2(p['bn1_g']), r2(p['bn1_b']),
                   p['eps1'].reshape(1, 1).astype(jnp.float32))
    x1flat = x1h.reshape(2 * N, D)
    a2h = _sc_agg_feat(x1flat, echunks, zeros).reshape(2, N, D)
    xl, x3 = _tc_mlp2(x1h, a2h, p['W2a'], r2(p['b2a']),
                      r2(p['bn2_g']), r2(p['bn2_b']),
                      p['Wl1'], r2(p['bl1']), p['Wx1'], r2(p['bx1']),
                      p['Wx2'], r2(p['bx2']), r2(p['lnx_g']), r2(p['lnx_b']),
                      p['eps2'].reshape(1, 1).astype(jnp.float32))

    tail = jnp.pad(adj[:, NMAIN:], ((0, 0), (0, WT - (N - NMAIN))))
    pairs = jnp.stack([pos_edge[0], pos_edge[1]], axis=1).ravel()
    posf = pos_edge.ravel()
    x3p = jnp.pad(x3, ((0, NPC - N), (0, 0)))
    cn, xi, xj = _sc_cn(adj, tail, pairs, posf, xl)
    xcn = _tc_cnmm(cn.reshape(Q, NPC), x3p)

    wl3 = jnp.pad(p['WL3'], ((0, 0), (0, D - p['WL3'].shape[1])))
    bl3 = jnp.pad(p['bL3'], (0, D - p['bL3'].shape[0])).reshape(1, D)
    args = (p['Wi1'], r2(p['bi1']), r2(p['lni_g']), r2(p['lni_b']),
            p['Wi2'], r2(p['bi2']),
            p['Wc1'], r2(p['bc1']), p['Wc2'], r2(p['bc2']),
            r2(p['lnc_g']), r2(p['lnc_b']), p['Wc3'], r2(p['bc3']),
            p['beta'].reshape(1, 1).astype(jnp.float32),
            p['WL1'], r2(p['bL1']), r2(p['lnL1_g']), r2(p['lnL1_b']),
            p['WL2'], r2(p['bL2']), r2(p['lnL2_g']), r2(p['lnL2_b']),
            wl3, bl3)
    o = _tc_final(xcn, xi, xj, args)
    return o[:, :7]
